# Initial kernel scaffold; baseline (speedup 1.0000x reference)
import jax
import jax.numpy as jnp
from jax.experimental import pallas as pl


def kernel(features, edge_index, W1, b1, W2, b2, Wc, bc):
    raise NotImplementedError("write your pallas kernel here")


# speedup vs baseline: 2.2063x; 2.2063x over previous
import functools

import jax
import jax.numpy as jnp
from jax import lax
from jax.experimental import pallas as pl
from jax.experimental.pallas import tpu as pltpu
from jax.experimental.pallas import tpu_sc as plsc

_NC = 2
_NS = 16
_NT = _NC * _NS
_B = 128
_L = 16
_BR = 1024


def _sc_mesh():
    return plsc.VectorSubcoreMesh(core_axis_name="c", subcore_axis_name="s")


def _build_hist(npad, nb):
    rows_pt = npad // _NS

    @functools.partial(
        pl.kernel, mesh=_sc_mesh(),
        out_type=(jax.ShapeDtypeStruct((_NC, npad, 16), jnp.float32),
                  jax.ShapeDtypeStruct((_NC, npad, 16), jnp.float32)),
        scratch_types=[
            pltpu.VMEM((nb, _B), jnp.int32),
            pltpu.VMEM((nb, _B), jnp.int32),
            pltpu.VMEM((_B, 16), jnp.float32),
            pltpu.VMEM_SHARED((npad, 16), jnp.float32),
            pltpu.VMEM_SHARED((npad, 16), jnp.float32),
        ])
    def hist(src_hbm, dst_hbm, ones_hbm, zrow_hbm, dsrc_hbm, ddst_hbm,
             src_v, dst_v, ones_v, dsrc_sh, ddst_sh):
        c = lax.axis_index("c")
        s = lax.axis_index("s")
        w = c * _NS + s
        pltpu.sync_copy(src_hbm.at[w], src_v)
        pltpu.sync_copy(dst_hbm.at[w], dst_v)
        pltpu.sync_copy(ones_hbm, ones_v)
        sl = pl.ds(s * rows_pt, rows_pt)
        pltpu.sync_copy(zrow_hbm, dsrc_sh.at[sl])
        pltpu.sync_copy(zrow_hbm, ddst_sh.at[sl])
        plsc.subcore_barrier()

        @pl.loop(0, nb)
        def _(j):
            pltpu.sync_copy(ones_v, dsrc_sh.at[src_v.at[j]], add=True)
            pltpu.sync_copy(ones_v, ddst_sh.at[dst_v.at[j]], add=True)

        plsc.subcore_barrier()
        pltpu.sync_copy(dsrc_sh.at[sl], dsrc_hbm.at[c, sl])
        pltpu.sync_copy(ddst_sh.at[sl], ddst_hbm.at[c, sl])

    return hist


def _build_prop(npad, nb, d):
    half = npad // 2
    hpad = half + 8
    rows_pt = half // _NS

    @functools.partial(
        pl.kernel, mesh=_sc_mesh(),
        out_type=jax.ShapeDtypeStruct((npad, d), jnp.float32),
        scratch_types=[
            pltpu.VMEM((2 * nb, _B), jnp.int32),
            pltpu.VMEM((2 * nb, _B), jnp.int32),
            pltpu.VMEM((2, _B, d), jnp.float32),
            pltpu.VMEM_SHARED((hpad, d), jnp.float32),
            pltpu.SemaphoreType.DMA,
            pltpu.SemaphoreType.DMA,
        ])
    def prop(h_hbm, src_hbm, dst_hbm, zpad_hbm, agg_hbm,
             src_v, dst_v, rows_v, agg_sh, sem0, sem1):
        c = lax.axis_index("c")
        s = lax.axis_index("s")
        pltpu.sync_copy(src_hbm.at[2 * s], src_v.at[pl.ds(0, nb)])
        pltpu.sync_copy(src_hbm.at[2 * s + 1], src_v.at[pl.ds(nb, nb)])
        pltpu.sync_copy(dst_hbm.at[2 * s], dst_v.at[pl.ds(0, nb)])
        pltpu.sync_copy(dst_hbm.at[2 * s + 1], dst_v.at[pl.ds(nb, nb)])

        lo = c * half

        @pl.loop(0, 2 * nb)
        def _(j):
            @pl.loop(0, _B // _L)
            def _(k):
                v = dst_v[j, pl.ds(k * _L, _L)] - lo
                ok = (v >= 0) & (v < half)
                dst_v[j, pl.ds(k * _L, _L)] = jnp.where(ok, v, half)

        pltpu.sync_copy(zpad_hbm, agg_sh.at[pl.ds(s * rows_pt, rows_pt)])

        @pl.when(s == _NS - 1)
        def _():
            pltpu.sync_copy(zpad_hbm.at[pl.ds(0, 8)],
                            agg_sh.at[pl.ds(half, 8)])

        plsc.subcore_barrier()

        @pl.loop(0, 2 * nb, step=2)
        def _(j):
            pltpu.async_copy(h_hbm.at[src_v.at[j]], rows_v.at[0],
                             sem0).wait()
            pltpu.sync_copy(rows_v.at[0], agg_sh.at[dst_v.at[j]], add=True)
            pltpu.async_copy(h_hbm.at[src_v.at[j + 1]], rows_v.at[1],
                             sem1).wait()
            pltpu.sync_copy(rows_v.at[1], agg_sh.at[dst_v.at[j + 1]],
                            add=True)

        plsc.subcore_barrier()
        pltpu.sync_copy(agg_sh.at[pl.ds(s * rows_pt, rows_pt)],
                        agg_hbm.at[pl.ds(lo + s * rows_pt, rows_pt)])

    return prop


def _norm_col(dref):
    d = dref[0, :, 0:1] + dref[1, :, 0:1]
    return jnp.where(d > 0, lax.rsqrt(d), 0.0)


def _dot(a, b):
    return jnp.dot(a, b, preferred_element_type=jnp.float32,
                   precision=lax.Precision.HIGHEST)


def _prescale_body(x_ref, dsrc_ref, o_ref):
    o_ref[...] = x_ref[...] * _norm_col(dsrc_ref)


def _mid_body(agg_ref, ddst_ref, dsrc_ref, w_ref, b_ref, o_ref):
    a = agg_ref[...] * _norm_col(ddst_ref)
    h = jnp.maximum(_dot(a, w_ref[...]) + b_ref[...], 0.0)
    o_ref[...] = h * _norm_col(dsrc_ref)


def _fin_body(agg_ref, ddst_ref, w_ref, b_ref, wc_ref, bc_ref, o_ref):
    a = agg_ref[...] * _norm_col(ddst_ref)
    h = jnp.maximum(_dot(a, w_ref[...]) + b_ref[...], 0.0)
    o_ref[...] = _dot(h, wc_ref[...]) + bc_ref[...]


def kernel(features, edge_index, W1, b1, W2, b2, Wc, bc):
    n, d = features.shape
    e = edge_index.shape[1]
    h = W1.shape[1]
    c_out = Wc.shape[1]
    npad = -(-(n + 1) // 2048) * 2048
    rows_pt = npad // _NS
    nb = -(-e // (_NT * _B))
    nb += nb % 2
    epad = _NT * nb * _B

    src = (jnp.full((epad,), n, jnp.int32).at[:e].set(edge_index[0])
           .reshape(_NT, nb, _B))
    dst = (jnp.full((epad,), n, jnp.int32).at[:e].set(edge_index[1])
           .reshape(_NT, nb, _B))
    featp = jnp.zeros((npad, d), jnp.float32).at[:n, :].set(features)
    ones16 = jnp.ones((_B, 16), jnp.float32)
    z16 = jnp.zeros((rows_pt, 16), jnp.float32)
    zd = jnp.zeros((npad // 2 // _NS, d), jnp.float32)

    hist = _build_hist(npad, nb)
    prop = _build_prop(npad, nb, d)
    dsrc, ddst = hist(src, dst, ones16, z16)

    grid = (npad // _BR,)
    deg_spec = pl.BlockSpec((_NC, _BR, 16), lambda i: (0, i, 0))
    row_spec = pl.BlockSpec((_BR, d), lambda i: (i, 0))
    w_spec = pl.BlockSpec((d, h), lambda i: (0, 0))
    b_spec = pl.BlockSpec((1, h), lambda i: (0, 0))

    h0 = pl.pallas_call(
        _prescale_body, grid=grid,
        in_specs=[row_spec, deg_spec],
        out_specs=row_spec,
        out_shape=jax.ShapeDtypeStruct((npad, d), jnp.float32),
    )(featp, dsrc)

    agg1 = prop(h0, src, dst, zd)

    h1 = pl.pallas_call(
        _mid_body, grid=grid,
        in_specs=[row_spec, deg_spec, deg_spec, w_spec, b_spec],
        out_specs=row_spec,
        out_shape=jax.ShapeDtypeStruct((npad, h), jnp.float32),
    )(agg1, ddst, dsrc, W1, b1.reshape(1, h))

    agg2 = prop(h1, src, dst, zd)

    out = pl.pallas_call(
        _fin_body, grid=grid,
        in_specs=[row_spec, deg_spec, w_spec, b_spec,
                  pl.BlockSpec((h, c_out), lambda i: (0, 0)),
                  pl.BlockSpec((1, c_out), lambda i: (0, 0))],
        out_specs=pl.BlockSpec((_BR, c_out), lambda i: (i, 0)),
        out_shape=jax.ShapeDtypeStruct((npad, c_out), jnp.float32),
    )(agg2, ddst, W2, b2.reshape(1, h), Wc, bc.reshape(1, c_out))

    return out[:n]


# speedup vs baseline: 2.2171x; 1.0049x over previous
import dataclasses
import functools

import jax
import jax.numpy as jnp
from jax import lax
from jax.experimental import pallas as pl
from jax.experimental.pallas import tpu as pltpu
from jax.experimental.pallas import tpu_sc as plsc

_NC = 2
_NS = 16
_NT = _NC * _NS
_B = 128
_L = 16
_BR = 1024


def _sc_mesh():
    return plsc.VectorSubcoreMesh(core_axis_name="c", subcore_axis_name="s")


def _no_layout_params():
    cp = pltpu.CompilerParams()
    if "needs_layout_passes" in pltpu.CompilerParams.__dataclass_fields__:
        cp = dataclasses.replace(cp, needs_layout_passes=False)
    return cp


def _build_hist(npad, nb):
    rows_pt = npad // _NS

    @functools.partial(
        pl.kernel, mesh=_sc_mesh(),
        compiler_params=_no_layout_params(),
        out_type=(jax.ShapeDtypeStruct((_NC, npad), jnp.float32),
                  jax.ShapeDtypeStruct((_NC, npad), jnp.float32)),
        scratch_types=[
            pltpu.VMEM((nb, _B), jnp.int32),
            pltpu.VMEM((nb, _B), jnp.int32),
            pltpu.VMEM((npad,), jnp.float32),
            pltpu.VMEM((npad,), jnp.float32),
            pltpu.VMEM((_NS, rows_pt), jnp.float32),
            pltpu.VMEM((rows_pt,), jnp.float32),
            pltpu.VMEM_SHARED((_NS, npad), jnp.float32),
        ])
    def hist(src_hbm, dst_hbm, dsrc_hbm, ddst_hbm,
             src_v, dst_v, hs_v, hd_v, red_v, out_v, stage_sh):
        c = lax.axis_index("c")
        s = lax.axis_index("s")
        w = c * _NS + s
        pltpu.sync_copy(src_hbm.at[w], src_v)
        pltpu.sync_copy(dst_hbm.at[w], dst_v)

        zeros = jnp.zeros((_L,), jnp.float32)
        ones = jnp.ones((_L,), jnp.float32)

        @pl.loop(0, npad // _L)
        def _(j):
            hs_v[pl.ds(j * _L, _L)] = zeros
            hd_v[pl.ds(j * _L, _L)] = zeros

        @pl.loop(0, nb)
        def _(j):
            @pl.loop(0, _B // _L)
            def _(k):
                plsc.addupdate_scatter(hs_v, [src_v[j, pl.ds(k * _L, _L)]],
                                       ones)
                plsc.addupdate_scatter(hd_v, [dst_v[j, pl.ds(k * _L, _L)]],
                                       ones)

        for hloc, out_hbm in ((hs_v, dsrc_hbm), (hd_v, ddst_hbm)):
            pltpu.sync_copy(hloc, stage_sh.at[s])
            plsc.subcore_barrier()
            for t in range(_NS):
                pltpu.sync_copy(stage_sh.at[t, pl.ds(s * rows_pt, rows_pt)],
                                red_v.at[t])

            @pl.loop(0, rows_pt // _L)
            def _(j):
                sl = pl.ds(j * _L, _L)
                acc = red_v[0, sl]
                for t in range(1, _NS):
                    acc = acc + red_v[t, sl]
                out_v[sl] = acc

            pltpu.sync_copy(out_v, out_hbm.at[c, pl.ds(s * rows_pt, rows_pt)])
            plsc.subcore_barrier()

    return hist


def _build_prop(npad, nb, d):
    half = npad // 2
    hpad = half + 8
    rows_pt = half // _NS

    @functools.partial(
        pl.kernel, mesh=_sc_mesh(),
        out_type=jax.ShapeDtypeStruct((npad, d), jnp.float32),
        scratch_types=[
            pltpu.VMEM((2 * nb, _B), jnp.int32),
            pltpu.VMEM((2 * nb, _B), jnp.int32),
            pltpu.VMEM((2, _B, d), jnp.float32),
            pltpu.VMEM_SHARED((hpad, d), jnp.float32),
            pltpu.SemaphoreType.DMA,
            pltpu.SemaphoreType.DMA,
        ])
    def prop(h_hbm, src_hbm, dst_hbm, zpad_hbm, agg_hbm,
             src_v, dst_v, rows_v, agg_sh, sem0, sem1):
        c = lax.axis_index("c")
        s = lax.axis_index("s")
        pltpu.sync_copy(src_hbm.at[2 * s], src_v.at[pl.ds(0, nb)])
        pltpu.sync_copy(src_hbm.at[2 * s + 1], src_v.at[pl.ds(nb, nb)])
        pltpu.sync_copy(dst_hbm.at[2 * s], dst_v.at[pl.ds(0, nb)])
        pltpu.sync_copy(dst_hbm.at[2 * s + 1], dst_v.at[pl.ds(nb, nb)])

        lo = c * half

        @pl.loop(0, 2 * nb)
        def _(j):
            @pl.loop(0, _B // _L)
            def _(k):
                v = dst_v[j, pl.ds(k * _L, _L)] - lo
                ok = (v >= 0) & (v < half)
                dst_v[j, pl.ds(k * _L, _L)] = jnp.where(ok, v, half)

        pltpu.sync_copy(zpad_hbm, agg_sh.at[pl.ds(s * rows_pt, rows_pt)])

        @pl.when(s == _NS - 1)
        def _():
            pltpu.sync_copy(zpad_hbm.at[pl.ds(0, 8)],
                            agg_sh.at[pl.ds(half, 8)])

        plsc.subcore_barrier()

        @pl.loop(0, 2 * nb, step=2)
        def _(j):
            pltpu.async_copy(h_hbm.at[src_v.at[j]], rows_v.at[0],
                             sem0).wait()
            pltpu.sync_copy(rows_v.at[0], agg_sh.at[dst_v.at[j]], add=True)
            pltpu.async_copy(h_hbm.at[src_v.at[j + 1]], rows_v.at[1],
                             sem1).wait()
            pltpu.sync_copy(rows_v.at[1], agg_sh.at[dst_v.at[j + 1]],
                            add=True)

        plsc.subcore_barrier()
        pltpu.sync_copy(agg_sh.at[pl.ds(s * rows_pt, rows_pt)],
                        agg_hbm.at[pl.ds(lo + s * rows_pt, rows_pt)])

    return prop


def _norm_col(dref):
    d = dref[0, :, 0:1] + dref[1, :, 0:1]
    return jnp.where(d > 0, lax.rsqrt(d), 0.0)


def _dot(a, b):
    return jnp.dot(a, b, preferred_element_type=jnp.float32,
                   precision=lax.Precision.HIGHEST)


def _prescale_body(x_ref, dsrc_ref, o_ref):
    o_ref[...] = x_ref[...] * _norm_col(dsrc_ref)


def _mid_body(agg_ref, ddst_ref, dsrc_ref, w_ref, b_ref, o_ref):
    a = agg_ref[...] * _norm_col(ddst_ref)
    h = jnp.maximum(_dot(a, w_ref[...]) + b_ref[...], 0.0)
    o_ref[...] = h * _norm_col(dsrc_ref)


def _fin_body(agg_ref, ddst_ref, w_ref, b_ref, wc_ref, bc_ref, o_ref):
    a = agg_ref[...] * _norm_col(ddst_ref)
    h = jnp.maximum(_dot(a, w_ref[...]) + b_ref[...], 0.0)
    o_ref[...] = _dot(h, wc_ref[...]) + bc_ref[...]


def kernel(features, edge_index, W1, b1, W2, b2, Wc, bc):
    n, d = features.shape
    e = edge_index.shape[1]
    h = W1.shape[1]
    c_out = Wc.shape[1]
    npad = -(-(n + 1) // 2048) * 2048
    rows_pt = npad // _NS
    nb = -(-e // (_NT * _B))
    nb += nb % 2
    epad = _NT * nb * _B

    src = (jnp.full((epad,), n, jnp.int32).at[:e].set(edge_index[0])
           .reshape(_NT, nb, _B))
    dst = (jnp.full((epad,), n, jnp.int32).at[:e].set(edge_index[1])
           .reshape(_NT, nb, _B))
    featp = jnp.zeros((npad, d), jnp.float32).at[:n, :].set(features)
    zd = jnp.zeros((npad // 2 // _NS, d), jnp.float32)

    hist = _build_hist(npad, nb)
    prop = _build_prop(npad, nb, d)
    dsrc, ddst = hist(src, dst)
    dsrc = dsrc.reshape(_NC, npad, 1)
    ddst = ddst.reshape(_NC, npad, 1)

    grid = (npad // _BR,)
    deg_spec = pl.BlockSpec((_NC, _BR, 1), lambda i: (0, i, 0))
    row_spec = pl.BlockSpec((_BR, d), lambda i: (i, 0))
    w_spec = pl.BlockSpec((d, h), lambda i: (0, 0))
    b_spec = pl.BlockSpec((1, h), lambda i: (0, 0))

    h0 = pl.pallas_call(
        _prescale_body, grid=grid,
        in_specs=[row_spec, deg_spec],
        out_specs=row_spec,
        out_shape=jax.ShapeDtypeStruct((npad, d), jnp.float32),
    )(featp, dsrc)

    agg1 = prop(h0, src, dst, zd)

    h1 = pl.pallas_call(
        _mid_body, grid=grid,
        in_specs=[row_spec, deg_spec, deg_spec, w_spec, b_spec],
        out_specs=row_spec,
        out_shape=jax.ShapeDtypeStruct((npad, h), jnp.float32),
    )(agg1, ddst, dsrc, W1, b1.reshape(1, h))

    agg2 = prop(h1, src, dst, zd)

    out = pl.pallas_call(
        _fin_body, grid=grid,
        in_specs=[row_spec, deg_spec, w_spec, b_spec,
                  pl.BlockSpec((h, c_out), lambda i: (0, 0)),
                  pl.BlockSpec((1, c_out), lambda i: (0, 0))],
        out_specs=pl.BlockSpec((_BR, c_out), lambda i: (i, 0)),
        out_shape=jax.ShapeDtypeStruct((npad, c_out), jnp.float32),
    )(agg2, ddst, W2, b2.reshape(1, h), Wc, bc.reshape(1, c_out))

    return out[:n]


# speedup vs baseline: 2.2271x; 1.0045x over previous
import dataclasses
import functools

import jax
import jax.numpy as jnp
from jax import lax
from jax.experimental import pallas as pl
from jax.experimental.pallas import tpu as pltpu
from jax.experimental.pallas import tpu_sc as plsc

_NC = 2
_NS = 16
_NT = _NC * _NS
_B = 128
_L = 16
_BR = 1024


def _sc_mesh():
    return plsc.VectorSubcoreMesh(core_axis_name="c", subcore_axis_name="s")


def _no_layout_params():
    cp = pltpu.CompilerParams()
    if "needs_layout_passes" in pltpu.CompilerParams.__dataclass_fields__:
        cp = dataclasses.replace(cp, needs_layout_passes=False)
    return cp


def _build_hist(npad, nb):
    rows_pt = npad // _NS

    @functools.partial(
        pl.kernel, mesh=_sc_mesh(),
        compiler_params=_no_layout_params(),
        out_type=(jax.ShapeDtypeStruct((_NC, npad), jnp.float32),
                  jax.ShapeDtypeStruct((_NC, npad), jnp.float32)),
        scratch_types=[
            pltpu.VMEM((nb, _B), jnp.int32),
            pltpu.VMEM((nb, _B), jnp.int32),
            pltpu.VMEM((npad,), jnp.float32),
            pltpu.VMEM((npad,), jnp.float32),
            pltpu.VMEM((_NS, rows_pt), jnp.float32),
            pltpu.VMEM((rows_pt,), jnp.float32),
            pltpu.VMEM_SHARED((_NS, npad), jnp.float32),
        ])
    def hist(src_hbm, dst_hbm, dsrc_hbm, ddst_hbm,
             src_v, dst_v, hs_v, hd_v, red_v, out_v, stage_sh):
        c = lax.axis_index("c")
        s = lax.axis_index("s")
        w = c * _NS + s
        pltpu.sync_copy(src_hbm.at[w], src_v)
        pltpu.sync_copy(dst_hbm.at[w], dst_v)

        zeros = jnp.zeros((_L,), jnp.float32)
        ones = jnp.ones((_L,), jnp.float32)

        @pl.loop(0, npad // _L)
        def _(j):
            hs_v[pl.ds(j * _L, _L)] = zeros
            hd_v[pl.ds(j * _L, _L)] = zeros

        @pl.loop(0, nb)
        def _(j):
            @pl.loop(0, _B // _L)
            def _(k):
                plsc.addupdate_scatter(hs_v, [src_v[j, pl.ds(k * _L, _L)]],
                                       ones)
                plsc.addupdate_scatter(hd_v, [dst_v[j, pl.ds(k * _L, _L)]],
                                       ones)

        for hloc, out_hbm in ((hs_v, dsrc_hbm), (hd_v, ddst_hbm)):
            pltpu.sync_copy(hloc, stage_sh.at[s])
            plsc.subcore_barrier()
            for t in range(_NS):
                pltpu.sync_copy(stage_sh.at[t, pl.ds(s * rows_pt, rows_pt)],
                                red_v.at[t])

            @pl.loop(0, rows_pt // _L)
            def _(j):
                sl = pl.ds(j * _L, _L)
                acc = red_v[0, sl]
                for t in range(1, _NS):
                    acc = acc + red_v[t, sl]
                out_v[sl] = acc

            pltpu.sync_copy(out_v, out_hbm.at[c, pl.ds(s * rows_pt, rows_pt)])
            plsc.subcore_barrier()

    return hist


def _build_prop(npad, nb, d):
    half = npad // 2
    hpad = half + 8
    rows_pt = half // _NS

    @functools.partial(
        pl.kernel, mesh=_sc_mesh(),
        out_type=jax.ShapeDtypeStruct((npad, d), jnp.float32),
        scratch_types=[
            pltpu.VMEM((2 * nb, _B), jnp.int32),
            pltpu.VMEM((2 * nb, _B), jnp.int32),
            pltpu.VMEM((2, _B, d), jnp.float32),
            pltpu.VMEM_SHARED((hpad, d), jnp.float32),
            pltpu.SemaphoreType.DMA,
            pltpu.SemaphoreType.DMA,
            pltpu.SemaphoreType.DMA,
            pltpu.SemaphoreType.DMA,
        ])
    def prop(h_hbm, src_hbm, dst_hbm, zpad_hbm, agg_hbm,
             src_v, dst_v, rows_v, agg_sh, g0, g1, s0, s1):
        c = lax.axis_index("c")
        s = lax.axis_index("s")
        pltpu.sync_copy(src_hbm.at[2 * s], src_v.at[pl.ds(0, nb)])
        pltpu.sync_copy(src_hbm.at[2 * s + 1], src_v.at[pl.ds(nb, nb)])
        pltpu.sync_copy(dst_hbm.at[2 * s], dst_v.at[pl.ds(0, nb)])
        pltpu.sync_copy(dst_hbm.at[2 * s + 1], dst_v.at[pl.ds(nb, nb)])

        lo = c * half

        @pl.loop(0, 2 * nb)
        def _(j):
            @pl.loop(0, _B // _L)
            def _(k):
                v = dst_v[j, pl.ds(k * _L, _L)] - lo
                ok = (v >= 0) & (v < half)
                dst_v[j, pl.ds(k * _L, _L)] = jnp.where(ok, v, half)

        pltpu.sync_copy(zpad_hbm, agg_sh.at[pl.ds(s * rows_pt, rows_pt)])

        @pl.when(s == _NS - 1)
        def _():
            pltpu.sync_copy(zpad_hbm.at[pl.ds(0, 8)],
                            agg_sh.at[pl.ds(half, 8)])

        plsc.subcore_barrier()

        last = 2 * nb - 1
        pltpu.async_copy(h_hbm.at[src_v.at[0]], rows_v.at[0], g0)
        pltpu.async_copy(h_hbm.at[src_v.at[1]], rows_v.at[1], g1)

        @pl.loop(0, 2 * nb, step=2)
        def _(j):
            pltpu.make_async_copy(h_hbm.at[src_v.at[j]], rows_v.at[0],
                                  g0).wait()
            pltpu.async_copy(rows_v.at[0], agg_sh.at[dst_v.at[j]], s0,
                             add=True)
            pltpu.make_async_copy(h_hbm.at[src_v.at[j + 1]], rows_v.at[1],
                                  g1).wait()
            pltpu.async_copy(rows_v.at[1], agg_sh.at[dst_v.at[j + 1]], s1,
                             add=True)
            jn0 = jnp.minimum(j + 2, last)
            jn1 = jnp.minimum(j + 3, last)
            pltpu.make_async_copy(rows_v.at[0], agg_sh.at[dst_v.at[j]],
                                  s0).wait()
            pltpu.async_copy(h_hbm.at[src_v.at[jn0]], rows_v.at[0], g0)
            pltpu.make_async_copy(rows_v.at[1], agg_sh.at[dst_v.at[j + 1]],
                                  s1).wait()
            pltpu.async_copy(h_hbm.at[src_v.at[jn1]], rows_v.at[1], g1)

        pltpu.make_async_copy(h_hbm.at[src_v.at[last]], rows_v.at[0],
                              g0).wait()
        pltpu.make_async_copy(h_hbm.at[src_v.at[last]], rows_v.at[1],
                              g1).wait()

        plsc.subcore_barrier()
        pltpu.sync_copy(agg_sh.at[pl.ds(s * rows_pt, rows_pt)],
                        agg_hbm.at[pl.ds(lo + s * rows_pt, rows_pt)])

    return prop


def _norm_col(dref):
    d = dref[0, :, 0:1] + dref[1, :, 0:1]
    return jnp.where(d > 0, lax.rsqrt(d), 0.0)


def _dot(a, b):
    return jnp.dot(a, b, preferred_element_type=jnp.float32,
                   precision=lax.Precision.HIGHEST)


def _prescale_body(x_ref, dsrc_ref, o_ref):
    o_ref[...] = x_ref[...] * _norm_col(dsrc_ref)


def _mid_body(agg_ref, ddst_ref, dsrc_ref, w_ref, b_ref, o_ref):
    a = agg_ref[...] * _norm_col(ddst_ref)
    h = jnp.maximum(_dot(a, w_ref[...]) + b_ref[...], 0.0)
    o_ref[...] = h * _norm_col(dsrc_ref)


def _fin_body(agg_ref, ddst_ref, w_ref, b_ref, wc_ref, bc_ref, o_ref):
    a = agg_ref[...] * _norm_col(ddst_ref)
    h = jnp.maximum(_dot(a, w_ref[...]) + b_ref[...], 0.0)
    o_ref[...] = _dot(h, wc_ref[...]) + bc_ref[...]


def kernel(features, edge_index, W1, b1, W2, b2, Wc, bc):
    n, d = features.shape
    e = edge_index.shape[1]
    h = W1.shape[1]
    c_out = Wc.shape[1]
    npad = -(-(n + 1) // 2048) * 2048
    rows_pt = npad // _NS
    nb = -(-e // (_NT * _B))
    nb += nb % 2
    epad = _NT * nb * _B

    src = (jnp.full((epad,), n, jnp.int32).at[:e].set(edge_index[0])
           .reshape(_NT, nb, _B))
    dst = (jnp.full((epad,), n, jnp.int32).at[:e].set(edge_index[1])
           .reshape(_NT, nb, _B))
    featp = jnp.zeros((npad, d), jnp.float32).at[:n, :].set(features)
    zd = jnp.zeros((npad // 2 // _NS, d), jnp.float32)

    hist = _build_hist(npad, nb)
    prop = jax.jit(_build_prop(npad, nb, d))
    dsrc, ddst = hist(src, dst)
    dsrc = dsrc.reshape(_NC, npad, 1)
    ddst = ddst.reshape(_NC, npad, 1)

    grid = (npad // _BR,)
    deg_spec = pl.BlockSpec((_NC, _BR, 1), lambda i: (0, i, 0))
    row_spec = pl.BlockSpec((_BR, d), lambda i: (i, 0))
    w_spec = pl.BlockSpec((d, h), lambda i: (0, 0))
    b_spec = pl.BlockSpec((1, h), lambda i: (0, 0))

    h0 = pl.pallas_call(
        _prescale_body, grid=grid,
        in_specs=[row_spec, deg_spec],
        out_specs=row_spec,
        out_shape=jax.ShapeDtypeStruct((npad, d), jnp.float32),
    )(featp, dsrc)

    agg1 = prop(h0, src, dst, zd)

    h1 = pl.pallas_call(
        _mid_body, grid=grid,
        in_specs=[row_spec, deg_spec, deg_spec, w_spec, b_spec],
        out_specs=row_spec,
        out_shape=jax.ShapeDtypeStruct((npad, h), jnp.float32),
    )(agg1, ddst, dsrc, W1, b1.reshape(1, h))

    agg2 = prop(h1, src, dst, zd)

    out = pl.pallas_call(
        _fin_body, grid=grid,
        in_specs=[row_spec, deg_spec, w_spec, b_spec,
                  pl.BlockSpec((h, c_out), lambda i: (0, 0)),
                  pl.BlockSpec((1, c_out), lambda i: (0, 0))],
        out_specs=pl.BlockSpec((_BR, c_out), lambda i: (i, 0)),
        out_shape=jax.ShapeDtypeStruct((npad, c_out), jnp.float32),
    )(agg2, ddst, W2, b2.reshape(1, h), Wc, bc.reshape(1, c_out))

    return out[:n]


# speedup vs baseline: 2.7049x; 1.2145x over previous
import dataclasses
import functools

import jax
import jax.numpy as jnp
from jax import lax
from jax.experimental import pallas as pl
from jax.experimental.pallas import tpu as pltpu
from jax.experimental.pallas import tpu_sc as plsc

_NC = 2
_NS = 16
_NT = _NC * _NS
_B = 128
_L = 16
_BR = 1024


def _sc_mesh():
    return plsc.VectorSubcoreMesh(core_axis_name="c", subcore_axis_name="s")


def _no_layout_params():
    cp = pltpu.CompilerParams()
    if "needs_layout_passes" in pltpu.CompilerParams.__dataclass_fields__:
        cp = dataclasses.replace(cp, needs_layout_passes=False)
    return cp


def _build_hist(npad, nb):
    rows_pt = npad // _NS
    half = npad // 2
    ne = nb * _B

    @functools.partial(
        pl.kernel, mesh=_sc_mesh(),
        compiler_params=_no_layout_params(),
        out_type=(jax.ShapeDtypeStruct((_NC, npad), jnp.float32),
                  jax.ShapeDtypeStruct((_NC, npad), jnp.float32),
                  jax.ShapeDtypeStruct((_NC, _NT, ne), jnp.int32),
                  jax.ShapeDtypeStruct((_NC, _NT, ne), jnp.int32),
                  jax.ShapeDtypeStruct((_NT, _L), jnp.int32)),
        scratch_types=[
            pltpu.VMEM((nb, _B), jnp.int32),
            pltpu.VMEM((nb, _B), jnp.int32),
            pltpu.VMEM((npad + _L,), jnp.float32),
            pltpu.VMEM((npad + _L,), jnp.float32),
            pltpu.VMEM((_NS, rows_pt), jnp.float32),
            pltpu.VMEM((rows_pt,), jnp.float32),
            pltpu.VMEM((ne + _L,), jnp.int32),
            pltpu.VMEM((ne + _L,), jnp.int32),
            pltpu.VMEM((ne + _L,), jnp.int32),
            pltpu.VMEM((ne + _L,), jnp.int32),
            pltpu.VMEM((_L,), jnp.int32),
            pltpu.VMEM_SHARED((_NS, npad), jnp.float32),
        ])
    def hist(src_hbm, dst_hbm, dsrc_hbm, ddst_hbm, srcc_hbm, dstc_hbm,
             nblk_hbm, src_v, dst_v, hs_v, hd_v, red_v, out_v,
             sc0_v, sc1_v, dc0_v, dc1_v, cnt_v, stage_sh):
        c = lax.axis_index("c")
        s = lax.axis_index("s")
        w = c * _NS + s
        pltpu.sync_copy(src_hbm.at[w], src_v)
        pltpu.sync_copy(dst_hbm.at[w], dst_v)

        zeros = jnp.zeros((_L,), jnp.float32)
        ones = jnp.ones((_L,), jnp.float32)
        n_dummy = jnp.full((_L,), npad - 8, jnp.int32)
        h_dummy = jnp.full((_L,), half, jnp.int32)

        @pl.loop(0, (npad + _L) // _L)
        def _(j):
            hs_v[pl.ds(j * _L, _L)] = zeros
            hd_v[pl.ds(j * _L, _L)] = zeros

        sc_b = (sc0_v, sc1_v)
        dc_b = (dc0_v, dc1_v)

        @pl.loop(0, (ne + _L) // _L)
        def _(j):
            for cc in range(_NC):
                sc_b[cc][pl.ds(j * _L, _L)] = n_dummy
                dc_b[cc][pl.ds(j * _L, _L)] = h_dummy

        @pl.loop(0, nb)
        def _(j):
            @pl.loop(0, _B // _L)
            def _(k):
                plsc.addupdate_scatter(hs_v, [src_v[j, pl.ds(k * _L, _L)]],
                                       ones)
                plsc.addupdate_scatter(hd_v, [dst_v[j, pl.ds(k * _L, _L)]],
                                       ones)

        def _cbody(i, offs):
            jj = i // (_B // _L)
            kk = i % (_B // _L)
            sv = src_v[jj, pl.ds(kk * _L, _L)]
            dv = dst_v[jj, pl.ds(kk * _L, _L)]
            new = []
            for cc in range(_NC):
                lv = dv - cc * half
                ok = (lv >= 0) & (lv < half)
                plsc.store_compressed(sc_b[cc].at[pl.ds(offs[cc], _L)],
                                      sv, mask=ok)
                plsc.store_compressed(dc_b[cc].at[pl.ds(offs[cc], _L)],
                                      lv, mask=ok)
                new.append(offs[cc] + jnp.sum(ok.astype(jnp.int32)))
            return tuple(new)

        offs = lax.fori_loop(0, nb * (_B // _L), _cbody,
                             tuple(jnp.int32(0) for _ in range(_NC)))

        iot = lax.iota(jnp.int32, _L)
        cvec = jnp.zeros((_L,), jnp.int32)
        for cc in range(_NC):
            nbl = jnp.maximum(2, (((offs[cc] + _B - 1) // _B) + 1) // 2 * 2)
            cvec = jnp.where(iot == cc, nbl, cvec)
        cnt_v[...] = cvec
        pltpu.sync_copy(cnt_v, nblk_hbm.at[w])
        for cc in range(_NC):
            pltpu.sync_copy(sc_b[cc].at[pl.ds(0, ne)], srcc_hbm.at[cc, w])
            pltpu.sync_copy(dc_b[cc].at[pl.ds(0, ne)], dstc_hbm.at[cc, w])

        for hloc, out_hbm in ((hs_v, dsrc_hbm), (hd_v, ddst_hbm)):
            pltpu.sync_copy(hloc.at[pl.ds(0, npad)], stage_sh.at[s])
            plsc.subcore_barrier()
            for t in range(_NS):
                pltpu.sync_copy(stage_sh.at[t, pl.ds(s * rows_pt, rows_pt)],
                                red_v.at[t])

            @pl.loop(0, rows_pt // _L)
            def _(j):
                sl = pl.ds(j * _L, _L)
                acc = red_v[0, sl]
                for t in range(1, _NS):
                    acc = acc + red_v[t, sl]
                out_v[sl] = acc

            pltpu.sync_copy(out_v, out_hbm.at[c, pl.ds(s * rows_pt, rows_pt)])
            plsc.subcore_barrier()

    return hist


def _build_prop(npad, nb, d):
    half = npad // 2
    hpad = half + 8
    rows_pt = half // _NS

    @functools.partial(
        pl.kernel, mesh=_sc_mesh(),
        compiler_params=_no_layout_params(),
        out_type=jax.ShapeDtypeStruct((npad, d), jnp.float32),
        scratch_types=[
            pltpu.VMEM((2 * nb, _B), jnp.int32),
            pltpu.VMEM((2 * nb, _B), jnp.int32),
            pltpu.VMEM((2, _B, d), jnp.float32),
            pltpu.VMEM((2, _L), jnp.int32),
            pltpu.VMEM_SHARED((hpad, d), jnp.float32),
            pltpu.SemaphoreType.DMA,
            pltpu.SemaphoreType.DMA,
            pltpu.SemaphoreType.DMA,
            pltpu.SemaphoreType.DMA,
        ])
    def prop(h_hbm, src_hbm, dst_hbm, nblk_hbm, zpad_hbm, agg_hbm,
             src_v, dst_v, rows_v, cnt_v, agg_sh, g0, g1, s0, s1):
        c = lax.axis_index("c")
        s = lax.axis_index("s")
        lo = c * half
        pltpu.sync_copy(nblk_hbm.at[2 * s], cnt_v.at[0])
        pltpu.sync_copy(nblk_hbm.at[2 * s + 1], cnt_v.at[1])
        iot = lax.iota(jnp.int32, _L)
        nb0 = jnp.sum(jnp.where(iot == c, cnt_v[0, :], 0))
        nb1 = jnp.sum(jnp.where(iot == c, cnt_v[1, :], 0))
        total = nb0 + nb1
        pltpu.sync_copy(src_hbm.at[c, 2 * s], src_v.at[pl.ds(0, nb)])
        pltpu.sync_copy(dst_hbm.at[c, 2 * s], dst_v.at[pl.ds(0, nb)])
        pltpu.sync_copy(src_hbm.at[c, 2 * s + 1], src_v.at[pl.ds(nb0, nb)])
        pltpu.sync_copy(dst_hbm.at[c, 2 * s + 1], dst_v.at[pl.ds(nb0, nb)])

        pltpu.sync_copy(zpad_hbm, agg_sh.at[pl.ds(s * rows_pt, rows_pt)])

        @pl.when(s == _NS - 1)
        def _():
            pltpu.sync_copy(zpad_hbm.at[pl.ds(0, 8)],
                            agg_sh.at[pl.ds(half, 8)])

        plsc.subcore_barrier()

        last = total - 1
        pltpu.async_copy(h_hbm.at[src_v.at[0]], rows_v.at[0], g0)
        pltpu.async_copy(h_hbm.at[src_v.at[1]], rows_v.at[1], g1)

        def _pbody(i, _):
            j = 2 * i
            pltpu.make_async_copy(h_hbm.at[src_v.at[j]], rows_v.at[0],
                                  g0).wait()
            pltpu.async_copy(rows_v.at[0], agg_sh.at[dst_v.at[j]], s0,
                             add=True)
            pltpu.make_async_copy(h_hbm.at[src_v.at[j + 1]], rows_v.at[1],
                                  g1).wait()
            pltpu.async_copy(rows_v.at[1], agg_sh.at[dst_v.at[j + 1]], s1,
                             add=True)
            jn0 = jnp.minimum(j + 2, last)
            jn1 = jnp.minimum(j + 3, last)
            pltpu.make_async_copy(rows_v.at[0], agg_sh.at[dst_v.at[j]],
                                  s0).wait()
            pltpu.async_copy(h_hbm.at[src_v.at[jn0]], rows_v.at[0], g0)
            pltpu.make_async_copy(rows_v.at[1], agg_sh.at[dst_v.at[j + 1]],
                                  s1).wait()
            pltpu.async_copy(h_hbm.at[src_v.at[jn1]], rows_v.at[1], g1)
            return 0

        lax.fori_loop(0, total // 2, _pbody, 0)

        pltpu.make_async_copy(h_hbm.at[src_v.at[last]], rows_v.at[0],
                              g0).wait()
        pltpu.make_async_copy(h_hbm.at[src_v.at[last]], rows_v.at[1],
                              g1).wait()

        plsc.subcore_barrier()
        pltpu.sync_copy(agg_sh.at[pl.ds(s * rows_pt, rows_pt)],
                        agg_hbm.at[pl.ds(lo + s * rows_pt, rows_pt)])

    return prop


def _norm_col(dref):
    d = dref[0, :, 0:1] + dref[1, :, 0:1]
    return jnp.where(d > 0, lax.rsqrt(d), 0.0)


def _dot(a, b):
    return jnp.dot(a, b, preferred_element_type=jnp.float32,
                   precision=lax.Precision.HIGHEST)


def _prescale_body(x_ref, dsrc_ref, o_ref):
    o_ref[...] = x_ref[...] * _norm_col(dsrc_ref)


def _mid_body(agg_ref, ddst_ref, dsrc_ref, w_ref, b_ref, o_ref):
    a = agg_ref[...] * _norm_col(ddst_ref)
    h = jnp.maximum(_dot(a, w_ref[...]) + b_ref[...], 0.0)
    o_ref[...] = h * _norm_col(dsrc_ref)


def _fin_body(agg_ref, ddst_ref, w_ref, b_ref, wc_ref, bc_ref, o_ref):
    a = agg_ref[...] * _norm_col(ddst_ref)
    h = jnp.maximum(_dot(a, w_ref[...]) + b_ref[...], 0.0)
    o_ref[...] = _dot(h, wc_ref[...]) + bc_ref[...]


def kernel(features, edge_index, W1, b1, W2, b2, Wc, bc):
    n, d = features.shape
    e = edge_index.shape[1]
    h = W1.shape[1]
    c_out = Wc.shape[1]
    npad = -(-(n + 1) // 2048) * 2048
    rows_pt = npad // _NS
    nb = -(-e // (_NT * _B))
    nb += nb % 2
    epad = _NT * nb * _B

    src = (jnp.full((epad,), npad, jnp.int32).at[:e].set(edge_index[0])
           .reshape(_NT, nb, _B))
    dst = (jnp.full((epad,), npad, jnp.int32).at[:e].set(edge_index[1])
           .reshape(_NT, nb, _B))
    featp = jnp.zeros((npad, d), jnp.float32).at[:n, :].set(features)
    zd = jnp.zeros((npad // 2 // _NS, d), jnp.float32)

    hist = _build_hist(npad, nb)
    prop = jax.jit(_build_prop(npad, nb, d))
    dsrc, ddst, srcc, dstc, nblk = hist(src, dst)
    dsrc = dsrc.reshape(_NC, npad, 1)
    ddst = ddst.reshape(_NC, npad, 1)
    srcc = srcc.reshape(_NC, _NT, nb, _B)
    dstc = dstc.reshape(_NC, _NT, nb, _B)

    grid = (npad // _BR,)
    deg_spec = pl.BlockSpec((_NC, _BR, 1), lambda i: (0, i, 0))
    row_spec = pl.BlockSpec((_BR, d), lambda i: (i, 0))
    w_spec = pl.BlockSpec((d, h), lambda i: (0, 0))
    b_spec = pl.BlockSpec((1, h), lambda i: (0, 0))

    h0 = pl.pallas_call(
        _prescale_body, grid=grid,
        in_specs=[row_spec, deg_spec],
        out_specs=row_spec,
        out_shape=jax.ShapeDtypeStruct((npad, d), jnp.float32),
    )(featp, dsrc)

    agg1 = prop(h0, srcc, dstc, nblk, zd)

    h1 = pl.pallas_call(
        _mid_body, grid=grid,
        in_specs=[row_spec, deg_spec, deg_spec, w_spec, b_spec],
        out_specs=row_spec,
        out_shape=jax.ShapeDtypeStruct((npad, h), jnp.float32),
    )(agg1, ddst, dsrc, W1, b1.reshape(1, h))

    agg2 = prop(h1, srcc, dstc, nblk, zd)

    out = pl.pallas_call(
        _fin_body, grid=grid,
        in_specs=[row_spec, deg_spec, w_spec, b_spec,
                  pl.BlockSpec((h, c_out), lambda i: (0, 0)),
                  pl.BlockSpec((1, c_out), lambda i: (0, 0))],
        out_specs=pl.BlockSpec((_BR, c_out), lambda i: (i, 0)),
        out_shape=jax.ShapeDtypeStruct((npad, c_out), jnp.float32),
    )(agg2, ddst, W2, b2.reshape(1, h), Wc, bc.reshape(1, c_out))

    return out[:n]
